# baseline (speedup 1.0000x reference)
import functools

import jax
import jax.numpy as jnp
from jax import lax
from jax.experimental import pallas as pl
from jax.experimental.pallas import tpu as pltpu
from jax.experimental.pallas import tpu_sc as plsc

_CB = 4


def _emb_lookup(idx2, embd, bsz, fld, d):
    tbs = bsz // 128
    n_blocks = fld * tbs
    n_workers = 32
    chunk = 128 * _CB
    per_w = n_blocks // n_workers // _CB
    nbuf = 3
    m_rows = fld * (d // 8) * tbs * 8
    mesh = plsc.VectorSubcoreMesh(core_axis_name="c", subcore_axis_name="s")

    scratch = (
        [pltpu.VMEM((chunk,), jnp.int32) for _ in range(nbuf)]
        + [pltpu.VMEM((chunk, d), jnp.float32) for _ in range(nbuf)]
        + [pltpu.VMEM((d // 8, _CB * 8, 128), jnp.float32) for _ in range(nbuf)]
        + [pltpu.SemaphoreType.DMA for _ in range(3 * nbuf)]
    )

    @functools.partial(
        pl.kernel,
        mesh=mesh,
        out_type=jax.ShapeDtypeStruct((m_rows, 128), jnp.float32),
        scratch_types=scratch,
        compiler_params=pltpu.CompilerParams(
            use_tc_tiling_on_sc=False, needs_layout_passes=False
        ),
    )
    def emb_kernel(idx_hbm, table_hbm, out2_hbm, *bufs):
        xi = bufs[:nbuf]
        gb = bufs[nbuf : 2 * nbuf]
        segb = bufs[2 * nbuf : 3 * nbuf]
        si = bufs[3 * nbuf : 4 * nbuf]
        sg = bufs[4 * nbuf : 5 * nbuf]
        so = bufs[5 * nbuf :]
        wid = lax.axis_index("s") * 2 + lax.axis_index("c")
        b0 = wid * per_w * _CB

        def blk(k):
            c = b0 + k * _CB
            f = lax.shift_right_logical(c, 7)
            tb = lax.bitwise_and(c, jnp.int32(127))
            return f, tb

        def idx_off(k):
            f, tb = blk(k)
            return f * bsz + tb * 128

        def out_row0(k, tj):
            f, tb = blk(k)
            return ((f * (d // 8) + tj) * tbs + tb) * 8

        def stage_idx(k, b):
            pltpu.async_copy(idx_hbm.at[pl.ds(idx_off(k), chunk)], xi[b], si[b])

        def wait_idx(k, b):
            pltpu.make_async_copy(
                idx_hbm.at[pl.ds(idx_off(k), chunk)], xi[b], si[b]
            ).wait()

        def start_gather(b):
            pltpu.async_copy(table_hbm.at[xi[b]], gb[b], sg[b])

        def wait_gather(b):
            pltpu.make_async_copy(table_hbm.at[xi[b]], gb[b], sg[b]).wait()

        def transpose(b):
            def tbody(g, carry):
                tbl = lax.shift_right_logical(g, 3)
                g16 = lax.bitwise_and(g, jnp.int32(7))
                r16 = lax.iota(jnp.int32, 16) + tbl * 128 + g16 * 16
                for j in range(d):
                    vals = plsc.load_gather(
                        gb[b], [r16, jnp.full((16,), j, jnp.int32)]
                    )
                    segb[b][j // 8, tbl * 8 + (j % 8), pl.ds(g16 * 16, 16)] = vals
                return carry

            lax.fori_loop(0, _CB * 8, tbody, 0)

        def start_out(k, b):
            for tj in range(d // 8):
                pltpu.async_copy(
                    segb[b].at[tj],
                    out2_hbm.at[pl.ds(out_row0(k, tj), _CB * 8)],
                    so[b],
                )

        def drain_out(k, b):
            for tj in range(d // 8):
                pltpu.make_async_copy(
                    segb[b].at[tj],
                    out2_hbm.at[pl.ds(out_row0(k, tj), _CB * 8)],
                    so[b],
                ).wait()

        for b in range(nbuf):
            stage_idx(b, b)
        wait_idx(0, 0)
        start_gather(0)
        wait_idx(1, 1)
        start_gather(1)

        n_groups = per_w // nbuf
        assert per_w % nbuf == 0 or per_w > nbuf

        def group(g, carry):
            for b in range(nbuf):
                k = g * nbuf + b
                b2 = (b + 2) % nbuf

                @pl.when(k + 2 < per_w)
                def _():
                    wait_idx(k + 2, b2)
                    start_gather(b2)

                @pl.when(k >= nbuf)
                def _():
                    drain_out(k - nbuf, b)

                wait_gather(b)
                transpose(b)
                start_out(k, b)

                @pl.when(k + nbuf < per_w)
                def _():
                    stage_idx(k + nbuf, b)

            return carry

        lax.fori_loop(0, n_groups, group, 0)
        rem = per_w - n_groups * nbuf
        for r in range(rem):
            k = n_groups * nbuf + r
            b = k % nbuf
            b2 = (b + 2) % nbuf
            if k + 2 < per_w:
                wait_idx(k + 2, b2)
                start_gather(b2)
            drain_out(k - nbuf, b)
            wait_gather(b)
            transpose(b)
            start_out(k, b)
        for k in range(per_w - nbuf, per_w):
            drain_out(k, k % nbuf)

    return emb_kernel(idx2, embd)


def kernel(x, embd):
    bsz, fld = x.shape
    v, d = embd.shape
    idx2 = x.T.reshape(bsz * fld)
    out2 = _emb_lookup(idx2, embd, bsz, fld, d)
    o = out2.reshape(fld, d // 8, bsz // 128, 8, 128)
    o = o.transpose(2, 4, 0, 1, 3)
    return o.reshape(bsz, fld, d)


# speedup vs baseline: 1.2651x; 1.2651x over previous
import functools

import jax
import jax.numpy as jnp
from jax import lax
from jax.experimental import pallas as pl
from jax.experimental.pallas import tpu as pltpu
from jax.experimental.pallas import tpu_sc as plsc

_CB = 4
_PITCH = 521


def _emb_lookup(idx2, embd, bsz, fld, d):
    tbs = bsz // 128
    n_blocks = fld * tbs
    n_workers = 32
    chunk = 128 * _CB
    per_w = n_blocks // n_workers // _CB
    nbuf = 2
    m_rows = fld * (d // 8) * tbs * 8
    mesh = plsc.VectorSubcoreMesh(core_axis_name="c", subcore_axis_name="s")

    scratch = (
        [pltpu.VMEM((chunk,), jnp.int32) for _ in range(nbuf)]
        + [pltpu.VMEM((chunk, d), jnp.float32) for _ in range(nbuf)]
        + [pltpu.VMEM((d, _PITCH), jnp.float32) for _ in range(nbuf)]
        + [pltpu.SemaphoreType.DMA for _ in range(3 * nbuf)]
    )

    @functools.partial(
        pl.kernel,
        mesh=mesh,
        out_type=jax.ShapeDtypeStruct((m_rows, 128), jnp.float32),
        scratch_types=scratch,
        compiler_params=pltpu.CompilerParams(
            use_tc_tiling_on_sc=False, needs_layout_passes=False
        ),
    )
    def emb_kernel(idx_hbm, table_hbm, out2_hbm, *bufs):
        xi = bufs[:nbuf]
        gb = bufs[nbuf : 2 * nbuf]
        segb = bufs[2 * nbuf : 3 * nbuf]
        si = bufs[3 * nbuf : 4 * nbuf]
        sg = bufs[4 * nbuf : 5 * nbuf]
        so = bufs[5 * nbuf :]
        wid = lax.axis_index("s") * 2 + lax.axis_index("c")
        b0 = wid * per_w * _CB

        def blk(k):
            c = b0 + k * _CB
            f = lax.shift_right_logical(c, 7)
            tb = lax.bitwise_and(c, jnp.int32(127))
            return f, tb

        def idx_off(k):
            f, tb = blk(k)
            return f * bsz + tb * 128

        def stage_idx(k, b):
            pltpu.async_copy(idx_hbm.at[pl.ds(idx_off(k), chunk)], xi[b], si[b])

        def wait_idx(k, b):
            pltpu.make_async_copy(
                idx_hbm.at[pl.ds(idx_off(k), chunk)], xi[b], si[b]
            ).wait()

        def start_gather(b):
            pltpu.async_copy(table_hbm.at[xi[b]], gb[b], sg[b])

        def wait_gather(b):
            pltpu.make_async_copy(table_hbm.at[xi[b]], gb[b], sg[b]).wait()

        def transpose(b):
            jv = lax.iota(jnp.int32, 16)
            zs = jnp.zeros((16,), jnp.int32)

            def tbody(r, carry):
                col = zs + r
                for h in range(d // 16):
                    vals = gb[b][r, pl.ds(16 * h, 16)]
                    plsc.store_scatter(segb[b], [jv + 16 * h, col], vals)
                return carry

            lax.fori_loop(0, chunk, tbody, 0, unroll=4)

        def out_slabs(k, b, make_only):
            f, tb = blk(k)
            for tj in range(d // 8):
                for tbl in range(_CB):
                    row0 = ((f * (d // 8) + tj) * tbs + tb + tbl) * 8
                    cp = pltpu.make_async_copy(
                        segb[b].at[pl.ds(tj * 8, 8), pl.ds(tbl * 128, 128)],
                        out2_hbm.at[pl.ds(row0, 8)],
                        so[b],
                    )
                    if make_only:
                        cp.wait()
                    else:
                        cp.start()

        for b in range(nbuf):
            stage_idx(b, b)
        wait_idx(0, 0)
        start_gather(0)

        n_groups = per_w // nbuf

        def group(g, carry):
            for b in range(nbuf):
                k = g * nbuf + b
                bn = (b + 1) % nbuf

                @pl.when(k + 1 < per_w)
                def _():
                    wait_idx(k + 1, bn)
                    start_gather(bn)

                @pl.when(k >= nbuf)
                def _():
                    out_slabs(k - nbuf, b, True)

                wait_gather(b)
                transpose(b)
                out_slabs(k, b, False)

                @pl.when(k + nbuf < per_w)
                def _():
                    stage_idx(k + nbuf, b)

            return carry

        lax.fori_loop(0, n_groups, group, 0)
        for k in range(per_w - nbuf, per_w):
            out_slabs(k, k % nbuf, True)

    return emb_kernel(idx2, embd)


def kernel(x, embd):
    bsz, fld = x.shape
    v, d = embd.shape
    idx2 = x.T.reshape(bsz * fld)
    out2 = _emb_lookup(idx2, embd, bsz, fld, d)
    o = out2.reshape(fld, d // 8, bsz // 128, 8, 128)
    o = o.transpose(2, 4, 0, 1, 3)
    return o.reshape(bsz, fld, d)
